# baseline (speedup 1.0000x reference)
import functools

import jax
import jax.numpy as jnp
from jax import lax
from jax.experimental import pallas as pl
from jax.experimental.pallas import tpu as pltpu
from jax.experimental.pallas import tpu_sc as plsc

N = 10000
NPAD = 10240
D = 128
NCORE = 2
SUB = 16
NW = NCORE * SUB
CHUNK = 125
ROWS_PER_SUB = NPAD // SUB
BR = 1024


def _sc_mesh():
    return plsc.VectorSubcoreMesh(core_axis_name="c", subcore_axis_name="s")


def _deg(dst3, ones1, zeros1):
    nchunk = dst3.shape[1]

    @functools.partial(
        pl.kernel,
        out_type=jax.ShapeDtypeStruct((NCORE * NPAD,), jnp.float32),
        mesh=_sc_mesh(),
        scratch_types=[
            pltpu.VMEM((nchunk, CHUNK), jnp.int32),
            pltpu.VMEM((CHUNK,), jnp.float32),
            pltpu.VMEM_SHARED((NPAD,), jnp.float32),
        ],
    )
    def deg_kernel(dst_hbm, ones_hbm, z_hbm, out_hbm, dst_v, ones_v, acc):
        cid = lax.axis_index("c")
        sid = lax.axis_index("s")
        wid = cid * SUB + sid
        pltpu.sync_copy(dst_hbm.at[wid], dst_v)
        pltpu.sync_copy(ones_hbm, ones_v)
        pltpu.sync_copy(z_hbm, acc.at[pl.ds(sid * ROWS_PER_SUB, ROWS_PER_SUB)])
        plsc.subcore_barrier()

        @pl.loop(0, nchunk)
        def _(j):
            pltpu.sync_copy(ones_v, acc.at[dst_v.at[j]], add=True)

        plsc.subcore_barrier()
        pltpu.sync_copy(
            acc.at[pl.ds(sid * ROWS_PER_SUB, ROWS_PER_SUB)],
            out_hbm.at[pl.ds(cid * NPAD + sid * ROWS_PER_SUB, ROWS_PER_SUB)],
        )

    return deg_kernel(dst3, ones1, zeros1)


def _agg(g, src2, dst2, zeros128):
    nchunk = src2.shape[0] // NW

    @functools.partial(
        pl.kernel,
        out_type=jax.ShapeDtypeStruct((NCORE * NPAD, D), jnp.float32),
        mesh=_sc_mesh(),
        scratch_types=[
            pltpu.VMEM((CHUNK,), jnp.int32),
            pltpu.VMEM((CHUNK,), jnp.int32),
            pltpu.VMEM((CHUNK,), jnp.int32),
            pltpu.VMEM((CHUNK,), jnp.int32),
            pltpu.VMEM((CHUNK, D), jnp.float32),
            pltpu.VMEM((CHUNK, D), jnp.float32),
            pltpu.VMEM_SHARED((NPAD, D), jnp.float32),
            pltpu.SemaphoreType.DMA,
            pltpu.SemaphoreType.DMA,
            pltpu.SemaphoreType.DMA,
            pltpu.SemaphoreType.DMA,
            pltpu.SemaphoreType.DMA,
            pltpu.SemaphoreType.DMA,
            pltpu.SemaphoreType.DMA,
            pltpu.SemaphoreType.DMA,
        ],
    )
    def agg_kernel(g_hbm, src_hbm, dst_hbm, z_hbm, out_hbm,
                   s0, s1, d0, d1, r0, r1, acc,
                   ss0, ss1, sd0, sd1, sg0, sg1, sc0, sc1):
        cid = lax.axis_index("c")
        sid = lax.axis_index("s")
        wid = cid * SUB + sid
        base = wid * nchunk
        sbuf, dbuf, rbuf = (s0, s1), (d0, d1), (r0, r1)
        ssem, dsem, gsem, csem = (ss0, ss1), (sd0, sd1), (sg0, sg1), (sc0, sc1)

        pltpu.async_copy(src_hbm.at[base], s0, ss0)
        pltpu.async_copy(dst_hbm.at[base], d0, sd0)
        pltpu.sync_copy(z_hbm, acc.at[pl.ds(sid * ROWS_PER_SUB, ROWS_PER_SUB)])
        pltpu.make_async_copy(src_hbm.at[base], s0, ss0).wait()
        pltpu.async_copy(g_hbm.at[s0], r0, sg0)
        pltpu.async_copy(src_hbm.at[base + 1], s1, ss1)
        plsc.subcore_barrier()

        @pl.loop(0, nchunk, step=2)
        def _(j):
            for b in range(2):
                jj = j + b
                o = 1 - b

                @pl.when(jj >= 1)
                def _():
                    pltpu.make_async_copy(rbuf[o], acc.at[dbuf[o]], csem[o]).wait()

                @pl.when(jj + 1 < nchunk)
                def _():
                    pltpu.make_async_copy(src_hbm.at[base + jj + 1], sbuf[o], ssem[o]).wait()
                    pltpu.async_copy(g_hbm.at[sbuf[o]], rbuf[o], gsem[o])
                    pltpu.async_copy(dst_hbm.at[base + jj + 1], dbuf[o], dsem[o])

                pltpu.make_async_copy(g_hbm.at[sbuf[b]], rbuf[b], gsem[b]).wait()

                @pl.when(jj + 2 < nchunk)
                def _():
                    pltpu.async_copy(src_hbm.at[base + jj + 2], sbuf[b], ssem[b])

                pltpu.make_async_copy(dst_hbm.at[base + jj], dbuf[b], dsem[b]).wait()
                pltpu.async_copy(rbuf[b], acc.at[dbuf[b]], csem[b], add=True)

        pltpu.make_async_copy(rbuf[(nchunk - 1) % 2], acc.at[dbuf[(nchunk - 1) % 2]],
                              csem[(nchunk - 1) % 2]).wait()
        plsc.subcore_barrier()
        pltpu.sync_copy(
            acc.at[pl.ds(sid * ROWS_PER_SUB, ROWS_PER_SUB)],
            out_hbm.at[pl.ds(cid * NPAD + sid * ROWS_PER_SUB, ROWS_PER_SUB)],
        )

    return agg_kernel(g, src2, dst2, zeros128)


def _dinv_block(dp_ref):
    deg = dp_ref[0] + dp_ref[1]
    return lax.rsqrt(jnp.maximum(deg, 1.0))


def _tc_mm(x, w0):
    def body(x_ref, w_ref, o_ref):
        o_ref[...] = jnp.dot(x_ref[...], w_ref[...], preferred_element_type=jnp.float32)

    return pl.pallas_call(
        body,
        grid=(NPAD // BR,),
        in_specs=[
            pl.BlockSpec((BR, D), lambda i: (i, 0)),
            pl.BlockSpec((D, D), lambda i: (0, 0)),
        ],
        out_specs=pl.BlockSpec((BR, D), lambda i: (i, 0)),
        out_shape=jax.ShapeDtypeStruct((NPAD, D), jnp.float32),
    )(x, w0)


def _tc_scale(ht, degp):
    def body(h_ref, dp_ref, o_ref):
        o_ref[...] = h_ref[...] * _dinv_block(dp_ref)

    return pl.pallas_call(
        body,
        grid=(NPAD // BR,),
        in_specs=[
            pl.BlockSpec((BR, D), lambda i: (i, 0)),
            pl.BlockSpec((2, BR, 1), lambda i: (0, i, 0)),
        ],
        out_specs=pl.BlockSpec((BR, D), lambda i: (i, 0)),
        out_shape=jax.ShapeDtypeStruct((NPAD, D), jnp.float32),
    )(ht, degp)


def _tc_mid(p2, degp, b0, w1):
    def body(p_ref, dp_ref, b_ref, w_ref, o_ref):
        d = _dinv_block(dp_ref)
        s = p_ref[0] + p_ref[1]
        h = jnp.tanh(s * d + b_ref[...])
        o_ref[...] = jnp.dot(h, w_ref[...], preferred_element_type=jnp.float32) * d

    return pl.pallas_call(
        body,
        grid=(NPAD // BR,),
        in_specs=[
            pl.BlockSpec((2, BR, D), lambda i: (0, i, 0)),
            pl.BlockSpec((2, BR, 1), lambda i: (0, i, 0)),
            pl.BlockSpec((1, D), lambda i: (0, 0)),
            pl.BlockSpec((D, D), lambda i: (0, 0)),
        ],
        out_specs=pl.BlockSpec((BR, D), lambda i: (i, 0)),
        out_shape=jax.ShapeDtypeStruct((NPAD, D), jnp.float32),
    )(p2, degp, b0, w1)


def _tc_fin(p2, degp, b1):
    def body(p_ref, dp_ref, b_ref, o_ref):
        d = _dinv_block(dp_ref)
        s = p_ref[0] + p_ref[1]
        o_ref[...] = jnp.tanh(s * d + b_ref[...])

    return pl.pallas_call(
        body,
        grid=(NPAD // BR,),
        in_specs=[
            pl.BlockSpec((2, BR, D), lambda i: (0, i, 0)),
            pl.BlockSpec((2, BR, 1), lambda i: (0, i, 0)),
            pl.BlockSpec((1, D), lambda i: (0, 0)),
        ],
        out_specs=pl.BlockSpec((BR, D), lambda i: (i, 0)),
        out_shape=jax.ShapeDtypeStruct((NPAD, D), jnp.float32),
    )(p2, degp, b1)


def kernel(x, edge_index_all, W0, b0, W1, b1):
    src2 = edge_index_all[0].reshape(-1, CHUNK)
    dst2 = edge_index_all[1].reshape(-1, CHUNK)
    dst3 = edge_index_all[1].reshape(NW, -1, CHUNK)
    zeros128 = jnp.zeros((ROWS_PER_SUB, D), jnp.float32)
    zeros1 = jnp.zeros((ROWS_PER_SUB,), jnp.float32)
    ones1 = jnp.ones((CHUNK,), jnp.float32)
    xpad = jnp.pad(x, ((0, NPAD - N), (0, 0)))

    degp = _deg(dst3, ones1, zeros1).reshape(NCORE, NPAD, 1)
    ht0 = _tc_mm(xpad, W0)
    g0 = _tc_scale(ht0, degp)
    p1 = _agg(g0, src2, dst2, zeros128).reshape(NCORE, NPAD, D)
    g1 = _tc_mid(p1, degp, b0.reshape(1, D), W1)
    p2 = _agg(g1, src2, dst2, zeros128).reshape(NCORE, NPAD, D)
    out = _tc_fin(p2, degp, b1.reshape(1, D))
    return out[:N]


# speedup vs baseline: 1.0216x; 1.0216x over previous
import functools

import jax
import jax.numpy as jnp
from jax import lax
from jax.experimental import pallas as pl
from jax.experimental.pallas import tpu as pltpu
from jax.experimental.pallas import tpu_sc as plsc

N = 10000
NPAD = 10240
D = 128
NCORE = 2
SUB = 16
NW = NCORE * SUB
CHUNK = 125
ROWS_PER_SUB = NPAD // SUB
BR = 1024


def _sc_mesh():
    return plsc.VectorSubcoreMesh(core_axis_name="c", subcore_axis_name="s")


def _deg(dst3, ones1, zeros1):
    nchunk = dst3.shape[1]

    @functools.partial(
        pl.kernel,
        out_type=jax.ShapeDtypeStruct((NCORE * NPAD,), jnp.float32),
        mesh=_sc_mesh(),
        scratch_types=[
            pltpu.VMEM((nchunk, CHUNK), jnp.int32),
            pltpu.VMEM((CHUNK,), jnp.float32),
            pltpu.VMEM_SHARED((NPAD,), jnp.float32),
        ],
    )
    def deg_kernel(dst_hbm, ones_hbm, z_hbm, out_hbm, dst_v, ones_v, acc):
        cid = lax.axis_index("c")
        sid = lax.axis_index("s")
        wid = cid * SUB + sid
        pltpu.sync_copy(dst_hbm.at[wid], dst_v)
        pltpu.sync_copy(ones_hbm, ones_v)
        pltpu.sync_copy(z_hbm, acc.at[pl.ds(sid * ROWS_PER_SUB, ROWS_PER_SUB)])
        plsc.subcore_barrier()

        @pl.loop(0, nchunk)
        def _(j):
            pltpu.sync_copy(ones_v, acc.at[dst_v.at[j]], add=True)

        plsc.subcore_barrier()
        pltpu.sync_copy(
            acc.at[pl.ds(sid * ROWS_PER_SUB, ROWS_PER_SUB)],
            out_hbm.at[pl.ds(cid * NPAD + sid * ROWS_PER_SUB, ROWS_PER_SUB)],
        )

    return deg_kernel(dst3, ones1, zeros1)


def _agg(g, src2, dst2, zeros128):
    nchunk = src2.shape[0] // NW

    @functools.partial(
        pl.kernel,
        out_type=jax.ShapeDtypeStruct((NCORE * NPAD, D), jnp.float32),
        mesh=_sc_mesh(),
        scratch_types=[
            pltpu.VMEM((CHUNK,), jnp.int32),
            pltpu.VMEM((CHUNK,), jnp.int32),
            pltpu.VMEM((CHUNK,), jnp.int32),
            pltpu.VMEM((CHUNK,), jnp.int32),
            pltpu.VMEM((CHUNK, D), jnp.float32),
            pltpu.VMEM((CHUNK, D), jnp.float32),
            pltpu.VMEM_SHARED((NPAD, D), jnp.float32),
            pltpu.SemaphoreType.DMA,
            pltpu.SemaphoreType.DMA,
            pltpu.SemaphoreType.DMA,
            pltpu.SemaphoreType.DMA,
            pltpu.SemaphoreType.DMA,
            pltpu.SemaphoreType.DMA,
            pltpu.SemaphoreType.DMA,
            pltpu.SemaphoreType.DMA,
        ],
    )
    def agg_kernel(g_hbm, src_hbm, dst_hbm, z_hbm, out_hbm,
                   s0, s1, d0, d1, r0, r1, acc,
                   ss0, ss1, sd0, sd1, sg0, sg1, sc0, sc1):
        cid = lax.axis_index("c")
        sid = lax.axis_index("s")
        wid = cid * SUB + sid
        base = wid * nchunk
        sbuf, dbuf, rbuf = (s0, s1), (d0, d1), (r0, r1)
        ssem, dsem, gsem, csem = (ss0, ss1), (sd0, sd1), (sg0, sg1), (sc0, sc1)

        pltpu.async_copy(src_hbm.at[base], s0, ss0)
        pltpu.async_copy(dst_hbm.at[base], d0, sd0)
        pltpu.sync_copy(z_hbm, acc.at[pl.ds(sid * ROWS_PER_SUB, ROWS_PER_SUB)])
        pltpu.make_async_copy(src_hbm.at[base], s0, ss0).wait()
        pltpu.async_copy(g_hbm.at[s0], r0, sg0)
        pltpu.async_copy(src_hbm.at[base + 1], s1, ss1)
        plsc.subcore_barrier()

        @pl.loop(0, nchunk, step=2)
        def _(j):
            for b in range(2):
                jj = j + b
                o = 1 - b

                @pl.when(jj >= 1)
                def _():
                    pltpu.make_async_copy(rbuf[o], acc.at[dbuf[o]], csem[o]).wait()

                @pl.when(jj + 1 < nchunk)
                def _():
                    pltpu.make_async_copy(src_hbm.at[base + jj + 1], sbuf[o], ssem[o]).wait()
                    pltpu.async_copy(g_hbm.at[sbuf[o]], rbuf[o], gsem[o])
                    pltpu.async_copy(dst_hbm.at[base + jj + 1], dbuf[o], dsem[o])

                pltpu.make_async_copy(g_hbm.at[sbuf[b]], rbuf[b], gsem[b]).wait()

                @pl.when(jj + 2 < nchunk)
                def _():
                    pltpu.async_copy(src_hbm.at[base + jj + 2], sbuf[b], ssem[b])

                pltpu.make_async_copy(dst_hbm.at[base + jj], dbuf[b], dsem[b]).wait()
                pltpu.async_copy(rbuf[b], acc.at[dbuf[b]], csem[b], add=True)

        pltpu.make_async_copy(rbuf[(nchunk - 1) % 2], acc.at[dbuf[(nchunk - 1) % 2]],
                              csem[(nchunk - 1) % 2]).wait()
        plsc.subcore_barrier()
        pltpu.sync_copy(
            acc.at[pl.ds(sid * ROWS_PER_SUB, ROWS_PER_SUB)],
            out_hbm.at[pl.ds(cid * NPAD + sid * ROWS_PER_SUB, ROWS_PER_SUB)],
        )

    return agg_kernel(g, src2, dst2, zeros128)


def _dinv_block(dp_ref):
    deg = dp_ref[0] + dp_ref[1]
    return lax.rsqrt(jnp.maximum(deg, 1.0))


def _tc_prep(degp, x, w0):
    def body(dp_ref, x_ref, w_ref, o_ref):
        d = _dinv_block(dp_ref)
        ht = jnp.dot(x_ref[...], w_ref[...], preferred_element_type=jnp.float32)
        o_ref[...] = ht * d

    return pl.pallas_call(
        body,
        grid=(NPAD // BR,),
        in_specs=[
            pl.BlockSpec((2, BR, 1), lambda i: (0, i, 0)),
            pl.BlockSpec((BR, D), lambda i: (i, 0)),
            pl.BlockSpec((D, D), lambda i: (0, 0)),
        ],
        out_specs=pl.BlockSpec((BR, D), lambda i: (i, 0)),
        out_shape=jax.ShapeDtypeStruct((NPAD, D), jnp.float32),
    )(degp, x, w0)


def _tc_mid(p2, degp, b0, w1):
    def body(p_ref, dp_ref, b_ref, w_ref, o_ref):
        d = _dinv_block(dp_ref)
        s = p_ref[0] + p_ref[1]
        h = jnp.tanh(s * d + b_ref[...])
        o_ref[...] = jnp.dot(h, w_ref[...], preferred_element_type=jnp.float32) * d

    return pl.pallas_call(
        body,
        grid=(NPAD // BR,),
        in_specs=[
            pl.BlockSpec((2, BR, D), lambda i: (0, i, 0)),
            pl.BlockSpec((2, BR, 1), lambda i: (0, i, 0)),
            pl.BlockSpec((1, D), lambda i: (0, 0)),
            pl.BlockSpec((D, D), lambda i: (0, 0)),
        ],
        out_specs=pl.BlockSpec((BR, D), lambda i: (i, 0)),
        out_shape=jax.ShapeDtypeStruct((NPAD, D), jnp.float32),
    )(p2, degp, b0, w1)


def _tc_fin(p2, degp, b1):
    def body(p_ref, dp_ref, b_ref, o_ref):
        d = _dinv_block(dp_ref)
        s = p_ref[0] + p_ref[1]
        o_ref[...] = jnp.tanh(s * d + b_ref[...])

    return pl.pallas_call(
        body,
        grid=(NPAD // BR,),
        in_specs=[
            pl.BlockSpec((2, BR, D), lambda i: (0, i, 0)),
            pl.BlockSpec((2, BR, 1), lambda i: (0, i, 0)),
            pl.BlockSpec((1, D), lambda i: (0, 0)),
        ],
        out_specs=pl.BlockSpec((BR, D), lambda i: (i, 0)),
        out_shape=jax.ShapeDtypeStruct((NPAD, D), jnp.float32),
    )(p2, degp, b1)


def kernel(x, edge_index_all, W0, b0, W1, b1):
    src2 = edge_index_all[0].reshape(-1, CHUNK)
    dst2 = edge_index_all[1].reshape(-1, CHUNK)
    dst3 = edge_index_all[1].reshape(NW, -1, CHUNK)
    zeros128 = jnp.zeros((ROWS_PER_SUB, D), jnp.float32)
    zeros1 = jnp.zeros((ROWS_PER_SUB,), jnp.float32)
    ones1 = jnp.ones((CHUNK,), jnp.float32)
    xpad = jnp.pad(x, ((0, NPAD - N), (0, 0)))

    degp = _deg(dst3, ones1, zeros1).reshape(NCORE, NPAD, 1)
    g0 = _tc_prep(degp, xpad, W0)
    p1 = _agg(g0, src2, dst2, zeros128).reshape(NCORE, NPAD, D)
    g1 = _tc_mid(p1, degp, b0.reshape(1, D), W1)
    p2 = _agg(g1, src2, dst2, zeros128).reshape(NCORE, NPAD, D)
    out = _tc_fin(p2, degp, b1.reshape(1, D))
    return out[:N]


# speedup vs baseline: 1.0333x; 1.0114x over previous
import functools

import jax
import jax.numpy as jnp
from jax import lax
from jax.experimental import pallas as pl
from jax.experimental.pallas import tpu as pltpu
from jax.experimental.pallas import tpu_sc as plsc

N = 10000
NPAD = 10240
D = 128
NCORE = 2
SUB = 16
NW = NCORE * SUB
CHUNK = 125
ROWS_PER_SUB = NPAD // SUB
BR = 1000


def _sc_mesh():
    return plsc.VectorSubcoreMesh(core_axis_name="c", subcore_axis_name="s")


def _deg(dst3, ones1, zeros1):
    nchunk = dst3.shape[1]

    @functools.partial(
        pl.kernel,
        out_type=jax.ShapeDtypeStruct((NCORE * NPAD,), jnp.float32),
        mesh=_sc_mesh(),
        scratch_types=[
            pltpu.VMEM((nchunk, CHUNK), jnp.int32),
            pltpu.VMEM((CHUNK,), jnp.float32),
            pltpu.VMEM_SHARED((NPAD,), jnp.float32),
        ],
    )
    def deg_kernel(dst_hbm, ones_hbm, z_hbm, out_hbm, dst_v, ones_v, acc):
        cid = lax.axis_index("c")
        sid = lax.axis_index("s")
        wid = cid * SUB + sid
        pltpu.sync_copy(dst_hbm.at[wid], dst_v)
        pltpu.sync_copy(ones_hbm, ones_v)
        pltpu.sync_copy(z_hbm, acc.at[pl.ds(sid * ROWS_PER_SUB, ROWS_PER_SUB)])
        plsc.subcore_barrier()

        @pl.loop(0, nchunk)
        def _(j):
            pltpu.sync_copy(ones_v, acc.at[dst_v.at[j]], add=True)

        plsc.subcore_barrier()
        pltpu.sync_copy(
            acc.at[pl.ds(sid * ROWS_PER_SUB, ROWS_PER_SUB)],
            out_hbm.at[pl.ds(cid * NPAD + sid * ROWS_PER_SUB, ROWS_PER_SUB)],
        )

    return deg_kernel(dst3, ones1, zeros1)


def _agg(g, src2, dst2, zeros128):
    nchunk = src2.shape[0] // NW

    @functools.partial(
        pl.kernel,
        out_type=jax.ShapeDtypeStruct((NCORE * NPAD, D), jnp.float32),
        mesh=_sc_mesh(),
        scratch_types=[
            pltpu.VMEM((CHUNK,), jnp.int32),
            pltpu.VMEM((CHUNK,), jnp.int32),
            pltpu.VMEM((CHUNK,), jnp.int32),
            pltpu.VMEM((CHUNK,), jnp.int32),
            pltpu.VMEM((CHUNK, D), jnp.float32),
            pltpu.VMEM((CHUNK, D), jnp.float32),
            pltpu.VMEM_SHARED((NPAD, D), jnp.float32),
            pltpu.SemaphoreType.DMA,
            pltpu.SemaphoreType.DMA,
            pltpu.SemaphoreType.DMA,
            pltpu.SemaphoreType.DMA,
            pltpu.SemaphoreType.DMA,
            pltpu.SemaphoreType.DMA,
            pltpu.SemaphoreType.DMA,
            pltpu.SemaphoreType.DMA,
        ],
    )
    def agg_kernel(g_hbm, src_hbm, dst_hbm, z_hbm, out_hbm,
                   s0, s1, d0, d1, r0, r1, acc,
                   ss0, ss1, sd0, sd1, sg0, sg1, sc0, sc1):
        cid = lax.axis_index("c")
        sid = lax.axis_index("s")
        wid = cid * SUB + sid
        base = wid * nchunk
        sbuf, dbuf, rbuf = (s0, s1), (d0, d1), (r0, r1)
        ssem, dsem, gsem, csem = (ss0, ss1), (sd0, sd1), (sg0, sg1), (sc0, sc1)

        pltpu.async_copy(src_hbm.at[base], s0, ss0)
        pltpu.async_copy(dst_hbm.at[base], d0, sd0)
        pltpu.sync_copy(z_hbm, acc.at[pl.ds(sid * ROWS_PER_SUB, ROWS_PER_SUB)])
        pltpu.make_async_copy(src_hbm.at[base], s0, ss0).wait()
        pltpu.async_copy(g_hbm.at[s0], r0, sg0)
        pltpu.async_copy(src_hbm.at[base + 1], s1, ss1)
        plsc.subcore_barrier()

        @pl.loop(0, nchunk, step=2)
        def _(j):
            for b in range(2):
                jj = j + b
                o = 1 - b

                @pl.when(jj >= 1)
                def _():
                    pltpu.make_async_copy(rbuf[o], acc.at[dbuf[o]], csem[o]).wait()

                @pl.when(jj + 1 < nchunk)
                def _():
                    pltpu.make_async_copy(src_hbm.at[base + jj + 1], sbuf[o], ssem[o]).wait()
                    pltpu.async_copy(g_hbm.at[sbuf[o]], rbuf[o], gsem[o])
                    pltpu.async_copy(dst_hbm.at[base + jj + 1], dbuf[o], dsem[o])

                pltpu.make_async_copy(g_hbm.at[sbuf[b]], rbuf[b], gsem[b]).wait()

                @pl.when(jj + 2 < nchunk)
                def _():
                    pltpu.async_copy(src_hbm.at[base + jj + 2], sbuf[b], ssem[b])

                pltpu.make_async_copy(dst_hbm.at[base + jj], dbuf[b], dsem[b]).wait()
                pltpu.async_copy(rbuf[b], acc.at[dbuf[b]], csem[b], add=True)

        pltpu.make_async_copy(rbuf[(nchunk - 1) % 2], acc.at[dbuf[(nchunk - 1) % 2]],
                              csem[(nchunk - 1) % 2]).wait()
        plsc.subcore_barrier()
        pltpu.sync_copy(
            acc.at[pl.ds(sid * ROWS_PER_SUB, ROWS_PER_SUB)],
            out_hbm.at[pl.ds(cid * NPAD + sid * ROWS_PER_SUB, ROWS_PER_SUB)],
        )

    return agg_kernel(g, src2, dst2, zeros128)


def _dinv_block(dp_ref):
    deg = dp_ref[0] + dp_ref[1]
    return lax.rsqrt(jnp.maximum(deg, 1.0))


def _tc_prep(degp, x, w0):
    def body(dp_ref, x_ref, w_ref, o_ref):
        d = _dinv_block(dp_ref)
        ht = jnp.dot(x_ref[...], w_ref[...], preferred_element_type=jnp.float32)
        o_ref[...] = ht * d

    return pl.pallas_call(
        body,
        grid=(N // BR,),
        in_specs=[
            pl.BlockSpec((2, BR, 1), lambda i: (0, i, 0)),
            pl.BlockSpec((BR, D), lambda i: (i, 0)),
            pl.BlockSpec((D, D), lambda i: (0, 0)),
        ],
        out_specs=pl.BlockSpec((BR, D), lambda i: (i, 0)),
        out_shape=jax.ShapeDtypeStruct((N, D), jnp.float32),
    )(degp, x, w0)


def _tc_mid(p2, degp, b0, w1):
    def body(p_ref, dp_ref, b_ref, w_ref, o_ref):
        d = _dinv_block(dp_ref)
        s = p_ref[0] + p_ref[1]
        h = jnp.tanh(s * d + b_ref[...])
        o_ref[...] = jnp.dot(h, w_ref[...], preferred_element_type=jnp.float32) * d

    return pl.pallas_call(
        body,
        grid=(N // BR,),
        in_specs=[
            pl.BlockSpec((2, BR, D), lambda i: (0, i, 0)),
            pl.BlockSpec((2, BR, 1), lambda i: (0, i, 0)),
            pl.BlockSpec((1, D), lambda i: (0, 0)),
            pl.BlockSpec((D, D), lambda i: (0, 0)),
        ],
        out_specs=pl.BlockSpec((BR, D), lambda i: (i, 0)),
        out_shape=jax.ShapeDtypeStruct((N, D), jnp.float32),
    )(p2, degp, b0, w1)


def _tc_fin(p2, degp, b1):
    def body(p_ref, dp_ref, b_ref, o_ref):
        d = _dinv_block(dp_ref)
        s = p_ref[0] + p_ref[1]
        o_ref[...] = jnp.tanh(s * d + b_ref[...])

    return pl.pallas_call(
        body,
        grid=(N // BR,),
        in_specs=[
            pl.BlockSpec((2, BR, D), lambda i: (0, i, 0)),
            pl.BlockSpec((2, BR, 1), lambda i: (0, i, 0)),
            pl.BlockSpec((1, D), lambda i: (0, 0)),
        ],
        out_specs=pl.BlockSpec((BR, D), lambda i: (i, 0)),
        out_shape=jax.ShapeDtypeStruct((N, D), jnp.float32),
    )(p2, degp, b1)


def kernel(x, edge_index_all, W0, b0, W1, b1):
    src2 = edge_index_all[0].reshape(-1, CHUNK)
    dst2 = edge_index_all[1].reshape(-1, CHUNK)
    dst3 = edge_index_all[1].reshape(NW, -1, CHUNK)
    zeros128 = jnp.zeros((ROWS_PER_SUB, D), jnp.float32)
    zeros1 = jnp.zeros((ROWS_PER_SUB,), jnp.float32)
    ones1 = jnp.ones((CHUNK,), jnp.float32)
    degp = _deg(dst3, ones1, zeros1).reshape(NCORE, NPAD, 1)
    g0 = _tc_prep(degp, x, W0)
    p1 = _agg(g0, src2, dst2, zeros128).reshape(NCORE, NPAD, D)
    g1 = _tc_mid(p1, degp, b0.reshape(1, D), W1)
    p2 = _agg(g1, src2, dst2, zeros128).reshape(NCORE, NPAD, D)
    return _tc_fin(p2, degp, b1.reshape(1, D))


# speedup vs baseline: 1.0448x; 1.0111x over previous
import functools

import jax
import jax.numpy as jnp
from jax import lax
from jax.experimental import pallas as pl
from jax.experimental.pallas import tpu as pltpu
from jax.experimental.pallas import tpu_sc as plsc

N = 10000
NPAD = 10240
D = 128
NCORE = 2
SUB = 16
NW = NCORE * SUB
CHUNK = 125
ROWS_PER_SUB = NPAD // SUB
BR = 1000


def _sc_mesh():
    return plsc.VectorSubcoreMesh(core_axis_name="c", subcore_axis_name="s")


def _deg(dst3, ones1, zeros1):
    nchunk = dst3.shape[1]

    @functools.partial(
        pl.kernel,
        out_type=jax.ShapeDtypeStruct((NCORE * NPAD,), jnp.float32),
        mesh=_sc_mesh(),
        scratch_types=[
            pltpu.VMEM((nchunk, CHUNK), jnp.int32),
            pltpu.VMEM((CHUNK,), jnp.float32),
            pltpu.VMEM_SHARED((NPAD,), jnp.float32),
        ],
    )
    def deg_kernel(dst_hbm, ones_hbm, z_hbm, out_hbm, dst_v, ones_v, acc):
        cid = lax.axis_index("c")
        sid = lax.axis_index("s")
        wid = cid * SUB + sid
        pltpu.sync_copy(dst_hbm.at[wid], dst_v)
        pltpu.sync_copy(ones_hbm, ones_v)
        pltpu.sync_copy(z_hbm, acc.at[pl.ds(sid * ROWS_PER_SUB, ROWS_PER_SUB)])
        plsc.subcore_barrier()

        @pl.loop(0, nchunk)
        def _(j):
            pltpu.sync_copy(ones_v, acc.at[dst_v.at[j]], add=True)

        plsc.subcore_barrier()
        pltpu.sync_copy(
            acc.at[pl.ds(sid * ROWS_PER_SUB, ROWS_PER_SUB)],
            out_hbm.at[pl.ds(cid * NPAD + sid * ROWS_PER_SUB, ROWS_PER_SUB)],
        )

    return deg_kernel(dst3, ones1, zeros1)


def _agg(g, idx3, zeros128):
    nchunk = idx3.shape[0] // NW

    @functools.partial(
        pl.kernel,
        out_type=jax.ShapeDtypeStruct((NCORE * NPAD, D), jnp.float32),
        mesh=_sc_mesh(),
        scratch_types=[
            pltpu.VMEM((2, CHUNK), jnp.int32),
            pltpu.VMEM((2, CHUNK), jnp.int32),
            pltpu.VMEM((2, CHUNK), jnp.int32),
            pltpu.VMEM((2, CHUNK), jnp.int32),
            pltpu.VMEM((CHUNK, D), jnp.float32),
            pltpu.VMEM((CHUNK, D), jnp.float32),
            pltpu.VMEM_SHARED((NPAD, D), jnp.float32),
            pltpu.SemaphoreType.DMA,
            pltpu.SemaphoreType.DMA,
            pltpu.SemaphoreType.DMA,
            pltpu.SemaphoreType.DMA,
            pltpu.SemaphoreType.DMA,
            pltpu.SemaphoreType.DMA,
            pltpu.SemaphoreType.DMA,
            pltpu.SemaphoreType.DMA,
        ],
    )
    def agg_kernel(g_hbm, idx_hbm, z_hbm, out_hbm,
                   i0, i1, i2, i3, r0, r1, acc,
                   si0, si1, si2, si3, sg0, sg1, sc0, sc1):
        cid = lax.axis_index("c")
        sid = lax.axis_index("s")
        wid = cid * SUB + sid
        base = wid * nchunk
        ibuf, rbuf = (i0, i1, i2, i3), (r0, r1)
        isem, gsem, csem = (si0, si1, si2, si3), (sg0, sg1), (sc0, sc1)

        pltpu.async_copy(idx_hbm.at[base], i0, si0)
        pltpu.sync_copy(z_hbm, acc.at[pl.ds(sid * ROWS_PER_SUB, ROWS_PER_SUB)])
        pltpu.make_async_copy(idx_hbm.at[base], i0, si0).wait()
        pltpu.async_copy(g_hbm.at[i0.at[0]], r0, sg0)
        pltpu.async_copy(idx_hbm.at[base + 1], i1, si1)
        plsc.subcore_barrier()

        @pl.loop(0, nchunk, step=4)
        def _(j):
            for b in range(4):
                jj = j + b
                rb = b % 2
                ro = 1 - rb

                @pl.when(jj >= 1)
                def _():
                    pltpu.make_async_copy(rbuf[ro], acc.at[ibuf[(b + 3) % 4].at[1]],
                                          csem[ro]).wait()

                @pl.when(jj + 1 < nchunk)
                def _():
                    pltpu.make_async_copy(idx_hbm.at[base + jj + 1], ibuf[(b + 1) % 4],
                                          isem[(b + 1) % 4]).wait()
                    pltpu.async_copy(g_hbm.at[ibuf[(b + 1) % 4].at[0]], rbuf[ro], gsem[ro])

                @pl.when(jj + 2 < nchunk)
                def _():
                    pltpu.async_copy(idx_hbm.at[base + jj + 2], ibuf[(b + 2) % 4],
                                     isem[(b + 2) % 4])

                pltpu.make_async_copy(g_hbm.at[ibuf[b].at[0]], rbuf[rb], gsem[rb]).wait()
                pltpu.async_copy(rbuf[rb], acc.at[ibuf[b].at[1]], csem[rb], add=True)

        pltpu.make_async_copy(rbuf[(nchunk - 1) % 2], acc.at[ibuf[(nchunk - 1) % 4].at[1]],
                              csem[(nchunk - 1) % 2]).wait()
        plsc.subcore_barrier()
        pltpu.sync_copy(
            acc.at[pl.ds(sid * ROWS_PER_SUB, ROWS_PER_SUB)],
            out_hbm.at[pl.ds(cid * NPAD + sid * ROWS_PER_SUB, ROWS_PER_SUB)],
        )

    return agg_kernel(g, idx3, zeros128)


def _dinv_block(dp_ref):
    deg = dp_ref[0] + dp_ref[1]
    return lax.rsqrt(jnp.maximum(deg, 1.0))


def _tc_prep(degp, x, w0):
    def body(dp_ref, x_ref, w_ref, o_ref):
        d = _dinv_block(dp_ref)
        ht = jnp.dot(x_ref[...], w_ref[...], preferred_element_type=jnp.float32)
        o_ref[...] = ht * d

    return pl.pallas_call(
        body,
        grid=(N // BR,),
        in_specs=[
            pl.BlockSpec((2, BR, 1), lambda i: (0, i, 0)),
            pl.BlockSpec((BR, D), lambda i: (i, 0)),
            pl.BlockSpec((D, D), lambda i: (0, 0)),
        ],
        out_specs=pl.BlockSpec((BR, D), lambda i: (i, 0)),
        out_shape=jax.ShapeDtypeStruct((N, D), jnp.float32),
    )(degp, x, w0)


def _tc_mid(p2, degp, b0, w1):
    def body(p_ref, dp_ref, b_ref, w_ref, o_ref):
        d = _dinv_block(dp_ref)
        s = p_ref[0] + p_ref[1]
        h = jnp.tanh(s * d + b_ref[...])
        o_ref[...] = jnp.dot(h, w_ref[...], preferred_element_type=jnp.float32) * d

    return pl.pallas_call(
        body,
        grid=(N // BR,),
        in_specs=[
            pl.BlockSpec((2, BR, D), lambda i: (0, i, 0)),
            pl.BlockSpec((2, BR, 1), lambda i: (0, i, 0)),
            pl.BlockSpec((1, D), lambda i: (0, 0)),
            pl.BlockSpec((D, D), lambda i: (0, 0)),
        ],
        out_specs=pl.BlockSpec((BR, D), lambda i: (i, 0)),
        out_shape=jax.ShapeDtypeStruct((N, D), jnp.float32),
    )(p2, degp, b0, w1)


def _tc_fin(p2, degp, b1):
    def body(p_ref, dp_ref, b_ref, o_ref):
        d = _dinv_block(dp_ref)
        s = p_ref[0] + p_ref[1]
        o_ref[...] = jnp.tanh(s * d + b_ref[...])

    return pl.pallas_call(
        body,
        grid=(N // BR,),
        in_specs=[
            pl.BlockSpec((2, BR, D), lambda i: (0, i, 0)),
            pl.BlockSpec((2, BR, 1), lambda i: (0, i, 0)),
            pl.BlockSpec((1, D), lambda i: (0, 0)),
        ],
        out_specs=pl.BlockSpec((BR, D), lambda i: (i, 0)),
        out_shape=jax.ShapeDtypeStruct((N, D), jnp.float32),
    )(p2, degp, b1)


def kernel(x, edge_index_all, W0, b0, W1, b1):
    idx3 = jnp.concatenate(
        [edge_index_all[0].reshape(-1, 1, CHUNK),
         edge_index_all[1].reshape(-1, 1, CHUNK)], axis=1)
    dst3 = edge_index_all[1].reshape(NW, -1, CHUNK)
    zeros128 = jnp.zeros((ROWS_PER_SUB, D), jnp.float32)
    zeros1 = jnp.zeros((ROWS_PER_SUB,), jnp.float32)
    ones1 = jnp.ones((CHUNK,), jnp.float32)
    degp = _deg(dst3, ones1, zeros1).reshape(NCORE, NPAD, 1)
    g0 = _tc_prep(degp, x, W0)
    p1 = _agg(g0, idx3, zeros128).reshape(NCORE, NPAD, D)
    g1 = _tc_mid(p1, degp, b0.reshape(1, D), W1)
    p2 = _agg(g1, idx3, zeros128).reshape(NCORE, NPAD, D)
    return _tc_fin(p2, degp, b1.reshape(1, D))


# speedup vs baseline: 1.0593x; 1.0140x over previous
import functools

import jax
import jax.numpy as jnp
from jax import lax
from jax.experimental import pallas as pl
from jax.experimental.pallas import tpu as pltpu
from jax.experimental.pallas import tpu_sc as plsc

N = 10000
NPAD = 10240
D = 128
NCORE = 2
SUB = 16
NW = NCORE * SUB
CHUNK = 125
ROWS_PER_SUB = NPAD // SUB
BR = 1000


def _sc_mesh():
    return plsc.VectorSubcoreMesh(core_axis_name="c", subcore_axis_name="s")


def _deg(dst3, ones1, zeros1):
    nchunk = dst3.shape[1]

    @functools.partial(
        pl.kernel,
        out_type=jax.ShapeDtypeStruct((NCORE * NPAD,), jnp.float32),
        mesh=_sc_mesh(),
        scratch_types=[
            pltpu.VMEM((nchunk, CHUNK), jnp.int32),
            pltpu.VMEM((CHUNK,), jnp.float32),
            pltpu.VMEM_SHARED((NPAD,), jnp.float32),
            pltpu.SemaphoreType.DMA,
            pltpu.SemaphoreType.DMA,
            pltpu.SemaphoreType.DMA,
            pltpu.SemaphoreType.DMA,
        ],
    )
    def deg_kernel(dst_hbm, ones_hbm, z_hbm, out_hbm, dst_v, ones_v, acc,
                   q0, q1, q2, q3):
        cid = lax.axis_index("c")
        sid = lax.axis_index("s")
        wid = cid * SUB + sid
        qsem = (q0, q1, q2, q3)
        pltpu.sync_copy(dst_hbm.at[wid], dst_v)
        pltpu.sync_copy(ones_hbm, ones_v)
        pltpu.sync_copy(z_hbm, acc.at[pl.ds(sid * ROWS_PER_SUB, ROWS_PER_SUB)])
        plsc.subcore_barrier()

        @pl.loop(0, nchunk, step=4)
        def _(j):
            for b in range(4):
                jj = j + b

                @pl.when(jj >= 4)
                def _():
                    pltpu.make_async_copy(ones_v, acc.at[dst_v.at[jj - 4]], qsem[b]).wait()

                pltpu.async_copy(ones_v, acc.at[dst_v.at[jj]], qsem[b], add=True)

        for b in range(4):
            pltpu.make_async_copy(ones_v, acc.at[dst_v.at[nchunk - 4 + b]], qsem[b]).wait()
        plsc.subcore_barrier()
        pltpu.sync_copy(
            acc.at[pl.ds(sid * ROWS_PER_SUB, ROWS_PER_SUB)],
            out_hbm.at[pl.ds(cid * NPAD + sid * ROWS_PER_SUB, ROWS_PER_SUB)],
        )

    return deg_kernel(dst3, ones1, zeros1)


def _agg(g, idx3, zeros128):
    nchunk = idx3.shape[0] // NW

    @functools.partial(
        pl.kernel,
        out_type=jax.ShapeDtypeStruct((NCORE * NPAD, D), jnp.float32),
        mesh=_sc_mesh(),
        scratch_types=[
            pltpu.VMEM((2, CHUNK), jnp.int32),
            pltpu.VMEM((2, CHUNK), jnp.int32),
            pltpu.VMEM((2, CHUNK), jnp.int32),
            pltpu.VMEM((2, CHUNK), jnp.int32),
            pltpu.VMEM((CHUNK, D), jnp.float32),
            pltpu.VMEM((CHUNK, D), jnp.float32),
            pltpu.VMEM_SHARED((NPAD, D), jnp.float32),
            pltpu.SemaphoreType.DMA,
            pltpu.SemaphoreType.DMA,
            pltpu.SemaphoreType.DMA,
            pltpu.SemaphoreType.DMA,
            pltpu.SemaphoreType.DMA,
            pltpu.SemaphoreType.DMA,
            pltpu.SemaphoreType.DMA,
            pltpu.SemaphoreType.DMA,
        ],
    )
    def agg_kernel(g_hbm, idx_hbm, z_hbm, out_hbm,
                   i0, i1, i2, i3, r0, r1, acc,
                   si0, si1, si2, si3, sg0, sg1, sc0, sc1):
        cid = lax.axis_index("c")
        sid = lax.axis_index("s")
        wid = cid * SUB + sid
        base = wid * nchunk
        ibuf, rbuf = (i0, i1, i2, i3), (r0, r1)
        isem, gsem, csem = (si0, si1, si2, si3), (sg0, sg1), (sc0, sc1)

        pltpu.async_copy(idx_hbm.at[base], i0, si0)
        pltpu.sync_copy(z_hbm, acc.at[pl.ds(sid * ROWS_PER_SUB, ROWS_PER_SUB)])
        pltpu.make_async_copy(idx_hbm.at[base], i0, si0).wait()
        pltpu.async_copy(g_hbm.at[i0.at[0]], r0, sg0)
        pltpu.async_copy(idx_hbm.at[base + 1], i1, si1)
        plsc.subcore_barrier()

        @pl.loop(0, nchunk, step=4)
        def _(j):
            for b in range(4):
                jj = j + b
                rb = b % 2
                ro = 1 - rb

                @pl.when(jj >= 1)
                def _():
                    pltpu.make_async_copy(rbuf[ro], acc.at[ibuf[(b + 3) % 4].at[1]],
                                          csem[ro]).wait()

                @pl.when(jj + 1 < nchunk)
                def _():
                    pltpu.make_async_copy(idx_hbm.at[base + jj + 1], ibuf[(b + 1) % 4],
                                          isem[(b + 1) % 4]).wait()
                    pltpu.async_copy(g_hbm.at[ibuf[(b + 1) % 4].at[0]], rbuf[ro], gsem[ro])

                @pl.when(jj + 2 < nchunk)
                def _():
                    pltpu.async_copy(idx_hbm.at[base + jj + 2], ibuf[(b + 2) % 4],
                                     isem[(b + 2) % 4])

                pltpu.make_async_copy(g_hbm.at[ibuf[b].at[0]], rbuf[rb], gsem[rb]).wait()
                pltpu.async_copy(rbuf[rb], acc.at[ibuf[b].at[1]], csem[rb], add=True)

        pltpu.make_async_copy(rbuf[(nchunk - 1) % 2], acc.at[ibuf[(nchunk - 1) % 4].at[1]],
                              csem[(nchunk - 1) % 2]).wait()
        plsc.subcore_barrier()
        pltpu.sync_copy(
            acc.at[pl.ds(sid * ROWS_PER_SUB, ROWS_PER_SUB)],
            out_hbm.at[pl.ds(cid * NPAD + sid * ROWS_PER_SUB, ROWS_PER_SUB)],
        )

    return agg_kernel(g, idx3, zeros128)


def _dinv_block(dp_ref):
    deg = dp_ref[0] + dp_ref[1]
    return lax.rsqrt(jnp.maximum(deg, 1.0))


def _tc_prep(degp, x, w0):
    def body(dp_ref, x_ref, w_ref, o_ref):
        d = _dinv_block(dp_ref)
        ht = jnp.dot(x_ref[...], w_ref[...], preferred_element_type=jnp.float32)
        o_ref[...] = ht * d

    return pl.pallas_call(
        body,
        grid=(N // BR,),
        in_specs=[
            pl.BlockSpec((2, BR, 1), lambda i: (0, i, 0)),
            pl.BlockSpec((BR, D), lambda i: (i, 0)),
            pl.BlockSpec((D, D), lambda i: (0, 0)),
        ],
        out_specs=pl.BlockSpec((BR, D), lambda i: (i, 0)),
        out_shape=jax.ShapeDtypeStruct((N, D), jnp.float32),
    )(degp, x, w0)


def _tc_mid(p2, degp, b0, w1):
    def body(p_ref, dp_ref, b_ref, w_ref, o_ref):
        d = _dinv_block(dp_ref)
        s = p_ref[0] + p_ref[1]
        h = jnp.tanh(s * d + b_ref[...])
        o_ref[...] = jnp.dot(h, w_ref[...], preferred_element_type=jnp.float32) * d

    return pl.pallas_call(
        body,
        grid=(N // BR,),
        in_specs=[
            pl.BlockSpec((2, BR, D), lambda i: (0, i, 0)),
            pl.BlockSpec((2, BR, 1), lambda i: (0, i, 0)),
            pl.BlockSpec((1, D), lambda i: (0, 0)),
            pl.BlockSpec((D, D), lambda i: (0, 0)),
        ],
        out_specs=pl.BlockSpec((BR, D), lambda i: (i, 0)),
        out_shape=jax.ShapeDtypeStruct((N, D), jnp.float32),
    )(p2, degp, b0, w1)


def _tc_fin(p2, degp, b1):
    def body(p_ref, dp_ref, b_ref, o_ref):
        d = _dinv_block(dp_ref)
        s = p_ref[0] + p_ref[1]
        o_ref[...] = jnp.tanh(s * d + b_ref[...])

    return pl.pallas_call(
        body,
        grid=(N // BR,),
        in_specs=[
            pl.BlockSpec((2, BR, D), lambda i: (0, i, 0)),
            pl.BlockSpec((2, BR, 1), lambda i: (0, i, 0)),
            pl.BlockSpec((1, D), lambda i: (0, 0)),
        ],
        out_specs=pl.BlockSpec((BR, D), lambda i: (i, 0)),
        out_shape=jax.ShapeDtypeStruct((N, D), jnp.float32),
    )(p2, degp, b1)


def kernel(x, edge_index_all, W0, b0, W1, b1):
    idx3 = jnp.concatenate(
        [edge_index_all[0].reshape(-1, 1, CHUNK),
         edge_index_all[1].reshape(-1, 1, CHUNK)], axis=1)
    dst3 = edge_index_all[1].reshape(NW, -1, CHUNK)
    zeros128 = jnp.zeros((ROWS_PER_SUB, D), jnp.float32)
    zeros1 = jnp.zeros((ROWS_PER_SUB,), jnp.float32)
    ones1 = jnp.ones((CHUNK,), jnp.float32)
    degp = _deg(dst3, ones1, zeros1).reshape(NCORE, NPAD, 1)
    g0 = _tc_prep(degp, x, W0)
    p1 = _agg(g0, idx3, zeros128).reshape(NCORE, NPAD, D)
    g1 = _tc_mid(p1, degp, b0.reshape(1, D), W1)
    p2 = _agg(g1, idx3, zeros128).reshape(NCORE, NPAD, D)
    return _tc_fin(p2, degp, b1.reshape(1, D))
